# baseline (speedup 1.0000x reference)
import functools

import jax
import jax.numpy as jnp
from jax import lax
from jax.experimental import pallas as pl
from jax.experimental.pallas import tpu as pltpu
from jax.experimental.pallas import tpu_sc as plsc

B, S, D, FF, E = 1, 2048, 768, 2048, 64
N = B * S
BLK = 128
G = N * 2 // BLK + E
P = G * BLK
CH = 128
NC, NS = 2, 16
NW = NC * NS
RPT = P // NW
GP = G + 8
GCH = 64
NGC = RPT // GCH
TPB = N // NW


def _routing_body(x_ref, gwt_ref, gum_ref, d1_ref, d2_ref, w1_ref, w2_ref,
                  be_ref):
    x = x_ref[...]
    logits = jnp.dot(x, gwt_ref[...], preferred_element_type=jnp.float32)
    m = jnp.max(logits, axis=-1, keepdims=True)
    eg = jnp.exp(logits - m)
    gates = eg / jnp.sum(eg, axis=-1, keepdims=True)
    iota_e = lax.broadcasted_iota(jnp.int32, (N, E), 1)
    gmax = jnp.max(gates, axis=-1, keepdims=True)
    idx1 = jnp.min(jnp.where(gates == gmax, iota_e, E), axis=-1)
    mask1 = iota_e == idx1[:, None]
    pert = jnp.where(mask1, -jnp.inf, logits + gum_ref[...])
    pmax = jnp.max(pert, axis=-1, keepdims=True)
    idx2 = jnp.min(jnp.where(pert == pmax, iota_e, E), axis=-1)
    mask2 = iota_e == idx2[:, None]
    g1 = jnp.sum(jnp.where(mask1, gates, 0.0), axis=-1)
    g2 = jnp.sum(jnp.where(mask2, gates, 0.0), axis=-1)
    den = jnp.maximum(g1 + g2, jnp.finfo(jnp.float32).eps)
    w1_ref[0, :] = g1 / den
    w2_ref[0, :] = g2 / den

    m1f = mask1.astype(jnp.float32)
    m2f = mask2.astype(jnp.float32)
    tri = (lax.broadcasted_iota(jnp.int32, (CH, CH), 0)
           >= lax.broadcasted_iota(jnp.int32, (CH, CH), 1)).astype(jnp.float32)

    def cumsum_tokens(mf):
        outs = []
        carry = jnp.zeros((1, E), jnp.float32)
        for k in range(N // CH):
            cs = jnp.dot(tri, mf[k * CH:(k + 1) * CH, :],
                         precision=lax.Precision.HIGHEST) + carry
            outs.append(cs)
            carry = cs[CH - 1:CH, :]
        return jnp.concatenate(outs, axis=0)

    cum1 = cumsum_tokens(m1f)
    cum2 = cumsum_tokens(m2f)
    c1 = cum1[N - 1:N, :]
    c2 = cum2[N - 1:N, :]
    r1 = jnp.sum(jnp.where(mask1, cum1, 0.0), axis=-1) - 1.0
    r2 = jnp.sum(jnp.where(mask2, cum2, 0.0), axis=-1) - 1.0
    cnt = c1 + c2
    nb = jnp.ceil(cnt * (1.0 / BLK))
    upper = (lax.broadcasted_iota(jnp.int32, (E, E), 0)
             <= lax.broadcasted_iota(jnp.int32, (E, E), 1)).astype(jnp.float32)
    cumnb = jnp.dot(nb, upper, precision=lax.Precision.HIGHEST)
    poff = (cumnb - nb) * float(BLK)
    dest1 = jnp.sum(jnp.where(mask1, poff, 0.0), axis=-1) + r1
    dest2 = jnp.sum(jnp.where(mask2, poff + c1, 0.0), axis=-1) + r2
    d1_ref[0, :] = jnp.round(dest1).astype(jnp.int32)
    d2_ref[0, :] = jnp.round(dest2).astype(jnp.int32)
    cumnb_i = jnp.round(cumnb).astype(jnp.int32)
    bio = lax.broadcasted_iota(jnp.int32, (GP, E), 0)
    bev = jnp.sum((bio >= cumnb_i).astype(jnp.int32), axis=-1)
    nb_tot = cumnb_i[:, E - 1]
    be_ref[0, :] = jnp.where(bio[:, 0] < G, jnp.minimum(bev, E - 1), nb_tot)


def _routing(x, gwt, gum):
    out_shape = (
        jax.ShapeDtypeStruct((1, N), jnp.int32),
        jax.ShapeDtypeStruct((1, N), jnp.int32),
        jax.ShapeDtypeStruct((1, N), jnp.float32),
        jax.ShapeDtypeStruct((1, N), jnp.float32),
        jax.ShapeDtypeStruct((1, GP), jnp.int32),
    )
    return pl.pallas_call(_routing_body, out_shape=out_shape)(x, gwt, gum)


APT = 2 * N // NS
ZPT = P // NS


def _dispatch_body(x_hbm, d1_hbm, d2_hbm, wa1_hbm, wa2_hbm, xs_hbm, sw_hbm,
                   dvec, tokv, wv, zbi, zbf, gwin, rowbuf,
                   g_sh, sw_sh, gs0, gs1, ws0, ws1):
    cid = lax.axis_index("c")
    sid = lax.axis_index("s")
    wid = cid * NS + sid
    iota16 = lax.broadcasted_iota(jnp.int32, (16,), 0)

    for k in range(ZPT // 16):
        zbi[pl.ds(k * 16, 16)] = (iota16 + (sid * ZPT + k * 16)) & (N - 1)
        zbf[pl.ds(k * 16, 16)] = jnp.zeros((16,), jnp.float32)
    pltpu.sync_copy(zbi, g_sh.at[pl.ds(sid * ZPT, ZPT)])
    pltpu.sync_copy(zbf, sw_sh.at[pl.ds(sid * ZPT, ZPT)])
    plsc.subcore_barrier()

    tbase = (sid % (NS // 2)) * APT

    @pl.when(sid < NS // 2)
    def _():
        for j in range(APT // 128):
            pltpu.sync_copy(d1_hbm.at[pl.ds(tbase + j * 128, 128)],
                            dvec.at[j])
            pltpu.sync_copy(wa1_hbm.at[pl.ds(tbase + j * 128, 128)],
                            wv.at[j])

    @pl.when(sid >= NS // 2)
    def _():
        for j in range(APT // 128):
            pltpu.sync_copy(d2_hbm.at[pl.ds(tbase + j * 128, 128)],
                            dvec.at[j])
            pltpu.sync_copy(wa2_hbm.at[pl.ds(tbase + j * 128, 128)],
                            wv.at[j])

    for j in range(APT // 128):
        for k in range(8):
            tokv[j, pl.ds(k * 16, 16)] = iota16 + (tbase + j * 128 + k * 16)
    for j in range(APT // 128):
        pltpu.sync_copy(tokv.at[j], g_sh.at[dvec.at[j]])
        pltpu.sync_copy(wv.at[j], sw_sh.at[dvec.at[j]])
    plsc.subcore_barrier()

    win = wid * RPT
    for ci in range(NGC):
        pltpu.sync_copy(g_sh.at[pl.ds(win + ci * GCH, GCH)], gwin.at[ci])
    pltpu.sync_copy(sw_sh.at[pl.ds(win, RPT)], sw_hbm.at[pl.ds(win, RPT)])
    gsem = (gs0, gs1)
    wsem = (ws0, ws1)
    gd = [None, None]
    wd = [None, None]
    gd[0] = pltpu.async_copy(x_hbm.at[gwin.at[0]], rowbuf.at[0], gs0)
    for ci in range(NGC):
        b = ci % 2
        gd[b].wait()
        if ci + 1 < NGC:
            if wd[1 - b] is not None:
                wd[1 - b].wait()
            gd[1 - b] = pltpu.async_copy(x_hbm.at[gwin.at[ci + 1]],
                                         rowbuf.at[1 - b], gsem[1 - b])
        wd[b] = pltpu.async_copy(rowbuf.at[b],
                                 xs_hbm.at[pl.ds(win + ci * GCH, GCH)],
                                 wsem[b])
    for b in range(2):
        if wd[b] is not None:
            wd[b].wait()


_SC_PARAMS = pltpu.CompilerParams(needs_layout_passes=False)


def _dispatch(x, dest1, dest2, wa1, wa2):
    mesh = plsc.VectorSubcoreMesh(core_axis_name="c", subcore_axis_name="s",
                                  num_cores=NC, num_subcores=NS)
    fn = pl.kernel(
        _dispatch_body,
        compiler_params=_SC_PARAMS,
        out_type=(jax.ShapeDtypeStruct((P, D), jnp.float32),
                  jax.ShapeDtypeStruct((P,), jnp.float32)),
        mesh=mesh,
        scratch_types=[
            pltpu.VMEM((APT // 128, 128), jnp.int32),
            pltpu.VMEM((APT // 128, 128), jnp.int32),
            pltpu.VMEM((APT // 128, 128), jnp.float32),
            pltpu.VMEM((ZPT,), jnp.int32),
            pltpu.VMEM((ZPT,), jnp.float32),
            pltpu.VMEM((NGC, GCH), jnp.int32),
            pltpu.VMEM((2, GCH, D), jnp.float32),
            pltpu.VMEM_SHARED((P,), jnp.int32),
            pltpu.VMEM_SHARED((P,), jnp.float32),
            pltpu.SemaphoreType.DMA,
            pltpu.SemaphoreType.DMA,
            pltpu.SemaphoreType.DMA,
            pltpu.SemaphoreType.DMA,
        ],
    )
    return fn(x, dest1, dest2, wa1, wa2)


def _ffn_body(be_ref, xs_ref, w1_ref, w2_ref, sw_ref, ys_ref):
    f = pl.program_id(1)

    @pl.when(pl.program_id(0) < be_ref[G])
    def _():
        xb = xs_ref[...]
        h = jnp.dot(xb, w1_ref[0], preferred_element_type=jnp.float32)
        h = 0.5 * h * (1.0 + lax.erf(h * (2.0 ** -0.5)))
        yb = jnp.dot(h, w2_ref[0], preferred_element_type=jnp.float32)
        yb = yb * sw_ref[0, 0, :][:, None]

        @pl.when(f == 0)
        def _():
            ys_ref[...] = yb

        @pl.when(f == 1)
        def _():
            ys_ref[...] += yb


def _ffn(be, xs, W1, W2, sw3):
    grid_spec = pltpu.PrefetchScalarGridSpec(
        num_scalar_prefetch=1,
        grid=(G, 2),
        in_specs=[
            pl.BlockSpec((BLK, D), lambda b, f, be: (b, 0)),
            pl.BlockSpec((1, D, FF // 2), lambda b, f, be: (be[b], 0, f)),
            pl.BlockSpec((1, FF // 2, D), lambda b, f, be: (be[b], f, 0)),
            pl.BlockSpec((1, 1, BLK), lambda b, f, be: (b, 0, 0)),
        ],
        out_specs=pl.BlockSpec((BLK, D), lambda b, f, be: (b, 0)),
    )
    return pl.pallas_call(
        _ffn_body, grid_spec=grid_spec,
        out_shape=jax.ShapeDtypeStruct((P, D), jnp.float32),
    )(be, xs, W1, W2, sw3)


def _combine_body(ys_hbm, d1_hbm, d2_hbm, out_hbm, d1v, d2v, rb1, rb2, sem):
    wid = lax.axis_index("s") * NC + lax.axis_index("c")
    base = wid * TPB
    pltpu.sync_copy(d1_hbm.at[pl.ds(base, TPB)], d1v)
    pltpu.sync_copy(d2_hbm.at[pl.ds(base, TPB)], d2v)
    cp1 = pltpu.async_copy(ys_hbm.at[d1v], rb1, sem)
    cp2 = pltpu.async_copy(ys_hbm.at[d2v], rb2, sem)
    cp1.wait()
    cp2.wait()

    def ab(i, carry):
        r = i // (D // 16)
        c = (i % (D // 16)) * 16
        rb1[r, pl.ds(c, 16)] = rb1[r, pl.ds(c, 16)] + rb2[r, pl.ds(c, 16)]
        return carry

    lax.fori_loop(0, TPB * (D // 16), ab, 0)
    pltpu.sync_copy(rb1, out_hbm.at[pl.ds(base, TPB)])


def _combine(ys, dest1, dest2):
    mesh = plsc.VectorSubcoreMesh(core_axis_name="c", subcore_axis_name="s",
                                  num_cores=NC, num_subcores=NS)
    fn = pl.kernel(
        _combine_body,
        compiler_params=_SC_PARAMS,
        out_type=jax.ShapeDtypeStruct((N, D), jnp.float32),
        mesh=mesh,
        scratch_types=[
            pltpu.VMEM((TPB,), jnp.int32),
            pltpu.VMEM((TPB,), jnp.int32),
            pltpu.VMEM((TPB, D), jnp.float32),
            pltpu.VMEM((TPB, D), jnp.float32),
            pltpu.SemaphoreType.DMA,
        ],
    )
    return fn(ys, dest1, dest2)


def kernel(hidden_states, gate_w, W1, W2):
    x = hidden_states.reshape(N, D)
    u = jax.random.uniform(jax.random.key(42), (N, E), dtype=jnp.float32)
    gum = -jnp.log(-jnp.log(jnp.clip(u, 1e-20, None)) + 1e-20)
    d1, d2, wa1, wa2, be = _routing(x, gate_w.T, gum)
    d1, d2 = d1.reshape(N), d2.reshape(N)
    wa1, wa2, be = wa1.reshape(N), wa2.reshape(N), be.reshape(GP)
    xs, sw = _dispatch(x, d1, d2, wa1, wa2)
    ys = _ffn(be, xs, W1, W2, sw.reshape(G, 1, BLK))
    out = _combine(ys, d1, d2)
    return out.reshape(B, S, D)


# speedup vs baseline: 1.3356x; 1.3356x over previous
import functools

import jax
import jax.numpy as jnp
from jax import lax
from jax.experimental import pallas as pl
from jax.experimental.pallas import tpu as pltpu
from jax.experimental.pallas import tpu_sc as plsc

B, S, D, FF, E = 1, 2048, 768, 2048, 64
N = B * S
BLK = 128
G = N * 2 // BLK + E
P = G * BLK
CH = 128
NC, NS = 2, 16
NW = NC * NS
RPT = P // NW
GP = G + 8
GCH = 64
NGC = RPT // GCH
TPB = N // NW


def _routing_body(x_ref, gwt_ref, gum_ref, d1_ref, d2_ref, w1_ref, w2_ref,
                  be_ref):
    x = x_ref[...]
    logits = jnp.dot(x, gwt_ref[...], preferred_element_type=jnp.float32)
    m = jnp.max(logits, axis=-1, keepdims=True)
    eg = jnp.exp(logits - m)
    gates = eg / jnp.sum(eg, axis=-1, keepdims=True)
    iota_e = lax.broadcasted_iota(jnp.int32, (N, E), 1)
    gmax = jnp.max(gates, axis=-1, keepdims=True)
    idx1 = jnp.min(jnp.where(gates == gmax, iota_e, E), axis=-1)
    mask1 = iota_e == idx1[:, None]
    pert = jnp.where(mask1, -jnp.inf, logits + gum_ref[...])
    pmax = jnp.max(pert, axis=-1, keepdims=True)
    idx2 = jnp.min(jnp.where(pert == pmax, iota_e, E), axis=-1)
    mask2 = iota_e == idx2[:, None]
    g1 = jnp.sum(jnp.where(mask1, gates, 0.0), axis=-1)
    g2 = jnp.sum(jnp.where(mask2, gates, 0.0), axis=-1)
    den = jnp.maximum(g1 + g2, jnp.finfo(jnp.float32).eps)
    w1_ref[0, :] = g1 / den
    w2_ref[0, :] = g2 / den

    m1f = mask1.astype(jnp.float32)
    m2f = mask2.astype(jnp.float32)
    tri = (lax.broadcasted_iota(jnp.int32, (CH, CH), 0)
           >= lax.broadcasted_iota(jnp.int32, (CH, CH), 1)).astype(jnp.float32)

    def cumsum_tokens(mf):
        outs = []
        carry = jnp.zeros((1, E), jnp.float32)
        for k in range(N // CH):
            cs = jnp.dot(tri, mf[k * CH:(k + 1) * CH, :],
                         precision=lax.Precision.HIGHEST) + carry
            outs.append(cs)
            carry = cs[CH - 1:CH, :]
        return jnp.concatenate(outs, axis=0)

    cum1 = cumsum_tokens(m1f)
    cum2 = cumsum_tokens(m2f)
    c1 = cum1[N - 1:N, :]
    c2 = cum2[N - 1:N, :]
    r1 = jnp.sum(jnp.where(mask1, cum1, 0.0), axis=-1) - 1.0
    r2 = jnp.sum(jnp.where(mask2, cum2, 0.0), axis=-1) - 1.0
    cnt = c1 + c2
    nb = jnp.ceil(cnt * (1.0 / BLK))
    upper = (lax.broadcasted_iota(jnp.int32, (E, E), 0)
             <= lax.broadcasted_iota(jnp.int32, (E, E), 1)).astype(jnp.float32)
    cumnb = jnp.dot(nb, upper, precision=lax.Precision.HIGHEST)
    poff = (cumnb - nb) * float(BLK)
    dest1 = jnp.sum(jnp.where(mask1, poff, 0.0), axis=-1) + r1
    dest2 = jnp.sum(jnp.where(mask2, poff + c1, 0.0), axis=-1) + r2
    d1_ref[0, :] = jnp.round(dest1).astype(jnp.int32)
    d2_ref[0, :] = jnp.round(dest2).astype(jnp.int32)
    cumnb_i = jnp.round(cumnb).astype(jnp.int32)
    bio = lax.broadcasted_iota(jnp.int32, (GP, E), 0)
    bev = jnp.sum((bio >= cumnb_i).astype(jnp.int32), axis=-1)
    nb_tot = cumnb_i[:, E - 1]
    be_ref[0, :] = jnp.where(bio[:, 0] < G, jnp.minimum(bev, E - 1), nb_tot)


def _routing(x, gwt, gum):
    out_shape = (
        jax.ShapeDtypeStruct((1, N), jnp.int32),
        jax.ShapeDtypeStruct((1, N), jnp.int32),
        jax.ShapeDtypeStruct((1, N), jnp.float32),
        jax.ShapeDtypeStruct((1, N), jnp.float32),
        jax.ShapeDtypeStruct((1, GP), jnp.int32),
    )
    return pl.pallas_call(_routing_body, out_shape=out_shape)(x, gwt, gum)


APT = 2 * N // NS
ZPT = P // NS


def _dispatch_body(x_hbm, d1_hbm, d2_hbm, wa1_hbm, wa2_hbm, xs_hbm, sw_hbm,
                   dvec, tokv, wv, zbi, zbf, gwin, rowbuf,
                   g_sh, sw_sh, gs0, gs1, ws0, ws1):
    cid = lax.axis_index("c")
    sid = lax.axis_index("s")
    wid = cid * NS + sid
    iota16 = lax.broadcasted_iota(jnp.int32, (16,), 0)

    for k in range(ZPT // 16):
        zbi[pl.ds(k * 16, 16)] = (iota16 + (sid * ZPT + k * 16)) & (N - 1)
        zbf[pl.ds(k * 16, 16)] = jnp.zeros((16,), jnp.float32)
    pltpu.sync_copy(zbi, g_sh.at[pl.ds(sid * ZPT, ZPT)])
    pltpu.sync_copy(zbf, sw_sh.at[pl.ds(sid * ZPT, ZPT)])
    plsc.subcore_barrier()

    tbase = (sid % (NS // 2)) * APT

    @pl.when(sid < NS // 2)
    def _():
        for j in range(APT // 128):
            pltpu.sync_copy(d1_hbm.at[pl.ds(tbase + j * 128, 128)],
                            dvec.at[j])
            pltpu.sync_copy(wa1_hbm.at[pl.ds(tbase + j * 128, 128)],
                            wv.at[j])

    @pl.when(sid >= NS // 2)
    def _():
        for j in range(APT // 128):
            pltpu.sync_copy(d2_hbm.at[pl.ds(tbase + j * 128, 128)],
                            dvec.at[j])
            pltpu.sync_copy(wa2_hbm.at[pl.ds(tbase + j * 128, 128)],
                            wv.at[j])

    for j in range(APT // 128):
        for k in range(8):
            tokv[j, pl.ds(k * 16, 16)] = iota16 + (tbase + j * 128 + k * 16)
    for j in range(APT // 128):
        pltpu.sync_copy(tokv.at[j], g_sh.at[dvec.at[j]])
        pltpu.sync_copy(wv.at[j], sw_sh.at[dvec.at[j]])
    plsc.subcore_barrier()

    win = wid * RPT
    for ci in range(NGC):
        pltpu.sync_copy(g_sh.at[pl.ds(win + ci * GCH, GCH)], gwin.at[ci])
    pltpu.sync_copy(sw_sh.at[pl.ds(win, RPT)], sw_hbm.at[pl.ds(win, RPT)])
    gsem = (gs0, gs1)
    wsem = (ws0, ws1)
    gd = [None, None]
    wd = [None, None]
    gd[0] = pltpu.async_copy(x_hbm.at[gwin.at[0]], rowbuf.at[0], gs0)
    for ci in range(NGC):
        b = ci % 2
        gd[b].wait()
        if ci + 1 < NGC:
            if wd[1 - b] is not None:
                wd[1 - b].wait()
            gd[1 - b] = pltpu.async_copy(x_hbm.at[gwin.at[ci + 1]],
                                         rowbuf.at[1 - b], gsem[1 - b])
        wd[b] = pltpu.async_copy(rowbuf.at[b],
                                 xs_hbm.at[pl.ds(win + ci * GCH, GCH)],
                                 wsem[b])
    for b in range(2):
        if wd[b] is not None:
            wd[b].wait()


_SC_PARAMS = pltpu.CompilerParams(needs_layout_passes=False)


def _dispatch(x, dest1, dest2, wa1, wa2):
    mesh = plsc.VectorSubcoreMesh(core_axis_name="c", subcore_axis_name="s",
                                  num_cores=NC, num_subcores=NS)
    fn = pl.kernel(
        _dispatch_body,
        compiler_params=_SC_PARAMS,
        out_type=(jax.ShapeDtypeStruct((P, D), jnp.float32),
                  jax.ShapeDtypeStruct((P,), jnp.float32)),
        mesh=mesh,
        scratch_types=[
            pltpu.VMEM((APT // 128, 128), jnp.int32),
            pltpu.VMEM((APT // 128, 128), jnp.int32),
            pltpu.VMEM((APT // 128, 128), jnp.float32),
            pltpu.VMEM((ZPT,), jnp.int32),
            pltpu.VMEM((ZPT,), jnp.float32),
            pltpu.VMEM((NGC, GCH), jnp.int32),
            pltpu.VMEM((2, GCH, D), jnp.float32),
            pltpu.VMEM_SHARED((P,), jnp.int32),
            pltpu.VMEM_SHARED((P,), jnp.float32),
            pltpu.SemaphoreType.DMA,
            pltpu.SemaphoreType.DMA,
            pltpu.SemaphoreType.DMA,
            pltpu.SemaphoreType.DMA,
        ],
    )
    return fn(x, dest1, dest2, wa1, wa2)


def _ffn_body(be_ref, xs_ref, w1_ref, w2_ref, sw_ref, ys_ref):
    @pl.when(pl.program_id(0) < be_ref[G])
    def _():
        xb = xs_ref[...]
        h = jnp.dot(xb, w1_ref[0], preferred_element_type=jnp.float32)
        h = 0.5 * h * (1.0 + lax.erf(h * (2.0 ** -0.5)))
        yb = jnp.dot(h, w2_ref[0], preferred_element_type=jnp.float32)
        ys_ref[...] = yb * sw_ref[0, 0, :][:, None]


def _ffn(be, xs, W1, W2, sw3):
    grid_spec = pltpu.PrefetchScalarGridSpec(
        num_scalar_prefetch=1,
        grid=(G,),
        in_specs=[
            pl.BlockSpec((BLK, D), lambda b, be: (b, 0)),
            pl.BlockSpec((1, D, FF), lambda b, be: (be[b], 0, 0)),
            pl.BlockSpec((1, FF, D), lambda b, be: (be[b], 0, 0)),
            pl.BlockSpec((1, 1, BLK), lambda b, be: (b, 0, 0)),
        ],
        out_specs=pl.BlockSpec((BLK, D), lambda b, be: (b, 0)),
    )
    return pl.pallas_call(
        _ffn_body, grid_spec=grid_spec,
        out_shape=jax.ShapeDtypeStruct((P, D), jnp.float32),
    )(be, xs, W1, W2, sw3)


def _combine_body(ys_hbm, d1_hbm, d2_hbm, out_hbm, d1v, d2v, rb1, rb2, sem):
    wid = lax.axis_index("s") * NC + lax.axis_index("c")
    base = wid * TPB
    pltpu.sync_copy(d1_hbm.at[pl.ds(base, TPB)], d1v)
    pltpu.sync_copy(d2_hbm.at[pl.ds(base, TPB)], d2v)
    cp1 = pltpu.async_copy(ys_hbm.at[d1v], rb1, sem)
    cp2 = pltpu.async_copy(ys_hbm.at[d2v], rb2, sem)
    cp1.wait()
    cp2.wait()

    def ab(i, carry):
        r = i // (D // 16)
        c = (i % (D // 16)) * 16
        rb1[r, pl.ds(c, 16)] = rb1[r, pl.ds(c, 16)] + rb2[r, pl.ds(c, 16)]
        return carry

    lax.fori_loop(0, TPB * (D // 16), ab, 0)
    pltpu.sync_copy(rb1, out_hbm.at[pl.ds(base, TPB)])


def _combine(ys, dest1, dest2):
    mesh = plsc.VectorSubcoreMesh(core_axis_name="c", subcore_axis_name="s",
                                  num_cores=NC, num_subcores=NS)
    fn = pl.kernel(
        _combine_body,
        compiler_params=_SC_PARAMS,
        out_type=jax.ShapeDtypeStruct((N, D), jnp.float32),
        mesh=mesh,
        scratch_types=[
            pltpu.VMEM((TPB,), jnp.int32),
            pltpu.VMEM((TPB,), jnp.int32),
            pltpu.VMEM((TPB, D), jnp.float32),
            pltpu.VMEM((TPB, D), jnp.float32),
            pltpu.SemaphoreType.DMA,
        ],
    )
    return fn(ys, dest1, dest2)


def kernel(hidden_states, gate_w, W1, W2):
    x = hidden_states.reshape(N, D)
    u = jax.random.uniform(jax.random.key(42), (N, E), dtype=jnp.float32)
    gum = -jnp.log(-jnp.log(jnp.clip(u, 1e-20, None)) + 1e-20)
    d1, d2, wa1, wa2, be = _routing(x, gate_w.T, gum)
    d1, d2 = d1.reshape(N), d2.reshape(N)
    wa1, wa2, be = wa1.reshape(N), wa2.reshape(N), be.reshape(GP)
    xs, sw = _dispatch(x, d1, d2, wa1, wa2)
    ys = _ffn(be, xs, W1, W2, sw.reshape(G, 1, BLK))
    out = _combine(ys, d1, d2)
    return out.reshape(B, S, D)


# speedup vs baseline: 1.3893x; 1.0402x over previous
import functools

import jax
import jax.numpy as jnp
from jax import lax
from jax.experimental import pallas as pl
from jax.experimental.pallas import tpu as pltpu
from jax.experimental.pallas import tpu_sc as plsc

B, S, D, FF, E = 1, 2048, 768, 2048, 64
N = B * S
BLK = 128
G = N * 2 // BLK + E
P = G * BLK
CH = 128
NC, NS = 2, 16
NW = NC * NS
RPT = P // NW
GP = G + 8
GCH = 64
NGC = RPT // GCH
TPB = N // NW


def _routing_body(x_ref, gwt_ref, gum_ref, d1_ref, d2_ref, w1_ref, w2_ref,
                  be_ref):
    x = x_ref[...]
    logits = jnp.dot(x, gwt_ref[...], preferred_element_type=jnp.float32)
    m = jnp.max(logits, axis=-1, keepdims=True)
    eg = jnp.exp(logits - m)
    gates = eg / jnp.sum(eg, axis=-1, keepdims=True)
    iota_e = lax.broadcasted_iota(jnp.int32, (N, E), 1)
    gmax = jnp.max(gates, axis=-1, keepdims=True)
    idx1 = jnp.min(jnp.where(gates == gmax, iota_e, E), axis=-1)
    mask1 = iota_e == idx1[:, None]
    pert = jnp.where(mask1, -jnp.inf, logits + gum_ref[...])
    pmax = jnp.max(pert, axis=-1, keepdims=True)
    idx2 = jnp.min(jnp.where(pert == pmax, iota_e, E), axis=-1)
    mask2 = iota_e == idx2[:, None]
    g1 = jnp.sum(jnp.where(mask1, gates, 0.0), axis=-1)
    g2 = jnp.sum(jnp.where(mask2, gates, 0.0), axis=-1)
    den = jnp.maximum(g1 + g2, jnp.finfo(jnp.float32).eps)
    w1_ref[0, :] = g1 / den
    w2_ref[0, :] = g2 / den

    m1f = mask1.astype(jnp.float32)
    m2f = mask2.astype(jnp.float32)
    tri = (lax.broadcasted_iota(jnp.int32, (CH, CH), 0)
           >= lax.broadcasted_iota(jnp.int32, (CH, CH), 1)).astype(jnp.float32)

    def cumsum_tokens(mf):
        outs = []
        carry = jnp.zeros((1, E), jnp.float32)
        for k in range(N // CH):
            cs = jnp.dot(tri, mf[k * CH:(k + 1) * CH, :],
                         precision=lax.Precision.HIGHEST) + carry
            outs.append(cs)
            carry = cs[CH - 1:CH, :]
        return jnp.concatenate(outs, axis=0)

    cum1 = cumsum_tokens(m1f)
    cum2 = cumsum_tokens(m2f)
    c1 = cum1[N - 1:N, :]
    c2 = cum2[N - 1:N, :]
    r1 = jnp.sum(jnp.where(mask1, cum1, 0.0), axis=-1) - 1.0
    r2 = jnp.sum(jnp.where(mask2, cum2, 0.0), axis=-1) - 1.0
    cnt = c1 + c2
    nb = jnp.ceil(cnt * (1.0 / BLK))
    upper = (lax.broadcasted_iota(jnp.int32, (E, E), 0)
             <= lax.broadcasted_iota(jnp.int32, (E, E), 1)).astype(jnp.float32)
    cumnb = jnp.dot(nb, upper, precision=lax.Precision.HIGHEST)
    poff = (cumnb - nb) * float(BLK)
    dest1 = jnp.sum(jnp.where(mask1, poff, 0.0), axis=-1) + r1
    dest2 = jnp.sum(jnp.where(mask2, poff + c1, 0.0), axis=-1) + r2
    d1_ref[0, :] = jnp.round(dest1).astype(jnp.int32)
    d2_ref[0, :] = jnp.round(dest2).astype(jnp.int32)
    cumnb_i = jnp.round(cumnb).astype(jnp.int32)
    bio = lax.broadcasted_iota(jnp.int32, (GP, E), 0)
    bev = jnp.sum((bio >= cumnb_i).astype(jnp.int32), axis=-1)
    nb_tot = cumnb_i[:, E - 1]
    be_ref[0, :] = jnp.where(bio[:, 0] < G, jnp.minimum(bev, E - 1), nb_tot)


def _routing(x, gwt, gum):
    out_shape = (
        jax.ShapeDtypeStruct((1, N), jnp.int32),
        jax.ShapeDtypeStruct((1, N), jnp.int32),
        jax.ShapeDtypeStruct((1, N), jnp.float32),
        jax.ShapeDtypeStruct((1, N), jnp.float32),
        jax.ShapeDtypeStruct((1, GP), jnp.int32),
    )
    return pl.pallas_call(_routing_body, out_shape=out_shape)(x, gwt, gum)


APT = 2 * N // NS
ZPT = P // NS


def _dispatch_body(x_hbm, d1_hbm, d2_hbm, wa1_hbm, wa2_hbm, xs_hbm, sw_hbm,
                   dvec, tokv, wv, zbi, zbf, gwin, rowbuf,
                   g_sh, sw_sh, gs0, gs1, ws0, ws1):
    cid = lax.axis_index("c")
    sid = lax.axis_index("s")
    wid = cid * NS + sid
    iota16 = lax.broadcasted_iota(jnp.int32, (16,), 0)

    for k in range(ZPT // 16):
        zbi[pl.ds(k * 16, 16)] = (iota16 + (sid * ZPT + k * 16)) & (N - 1)
        zbf[pl.ds(k * 16, 16)] = jnp.zeros((16,), jnp.float32)
    pltpu.sync_copy(zbi, g_sh.at[pl.ds(sid * ZPT, ZPT)])
    pltpu.sync_copy(zbf, sw_sh.at[pl.ds(sid * ZPT, ZPT)])
    plsc.subcore_barrier()

    tbase = (sid % (NS // 2)) * APT

    @pl.when(sid < NS // 2)
    def _():
        for j in range(APT // 128):
            pltpu.sync_copy(d1_hbm.at[pl.ds(tbase + j * 128, 128)],
                            dvec.at[j])
            pltpu.sync_copy(wa1_hbm.at[pl.ds(tbase + j * 128, 128)],
                            wv.at[j])

    @pl.when(sid >= NS // 2)
    def _():
        for j in range(APT // 128):
            pltpu.sync_copy(d2_hbm.at[pl.ds(tbase + j * 128, 128)],
                            dvec.at[j])
            pltpu.sync_copy(wa2_hbm.at[pl.ds(tbase + j * 128, 128)],
                            wv.at[j])

    for j in range(APT // 128):
        for k in range(8):
            tokv[j, pl.ds(k * 16, 16)] = iota16 + (tbase + j * 128 + k * 16)
    for j in range(APT // 128):
        pltpu.sync_copy(tokv.at[j], g_sh.at[dvec.at[j]])
        pltpu.sync_copy(wv.at[j], sw_sh.at[dvec.at[j]])
    plsc.subcore_barrier()

    win = wid * RPT
    for ci in range(NGC):
        pltpu.sync_copy(g_sh.at[pl.ds(win + ci * GCH, GCH)], gwin.at[ci])
    pltpu.sync_copy(sw_sh.at[pl.ds(win, RPT)], sw_hbm.at[pl.ds(win, RPT)])
    gsem = (gs0, gs1)
    wsem = (ws0, ws1)
    gd = [None, None]
    wd = [None, None]
    gd[0] = pltpu.async_copy(x_hbm.at[gwin.at[0]], rowbuf.at[0], gs0)
    for ci in range(NGC):
        b = ci % 2
        gd[b].wait()
        if ci + 1 < NGC:
            if wd[1 - b] is not None:
                wd[1 - b].wait()
            gd[1 - b] = pltpu.async_copy(x_hbm.at[gwin.at[ci + 1]],
                                         rowbuf.at[1 - b], gsem[1 - b])
        wd[b] = pltpu.async_copy(rowbuf.at[b],
                                 xs_hbm.at[pl.ds(win + ci * GCH, GCH)],
                                 wsem[b])
    for b in range(2):
        if wd[b] is not None:
            wd[b].wait()


_SC_PARAMS = pltpu.CompilerParams(needs_layout_passes=False)


def _dispatch(x, dest1, dest2, wa1, wa2):
    mesh = plsc.VectorSubcoreMesh(core_axis_name="c", subcore_axis_name="s",
                                  num_cores=NC, num_subcores=NS)
    fn = pl.kernel(
        _dispatch_body,
        compiler_params=_SC_PARAMS,
        out_type=(jax.ShapeDtypeStruct((P, D), jnp.float32),
                  jax.ShapeDtypeStruct((P,), jnp.float32)),
        mesh=mesh,
        scratch_types=[
            pltpu.VMEM((APT // 128, 128), jnp.int32),
            pltpu.VMEM((APT // 128, 128), jnp.int32),
            pltpu.VMEM((APT // 128, 128), jnp.float32),
            pltpu.VMEM((ZPT,), jnp.int32),
            pltpu.VMEM((ZPT,), jnp.float32),
            pltpu.VMEM((NGC, GCH), jnp.int32),
            pltpu.VMEM((2, GCH, D), jnp.float32),
            pltpu.VMEM_SHARED((P,), jnp.int32),
            pltpu.VMEM_SHARED((P,), jnp.float32),
            pltpu.SemaphoreType.DMA,
            pltpu.SemaphoreType.DMA,
            pltpu.SemaphoreType.DMA,
            pltpu.SemaphoreType.DMA,
        ],
    )
    return fn(x, dest1, dest2, wa1, wa2)


def _ffn_body(be_ref, xs_ref, w1_ref, w2_ref, sw_ref, ys_ref):
    @pl.when(pl.program_id(0) < be_ref[G])
    def _():
        xb = xs_ref[...]
        h = jnp.dot(xb, w1_ref[0], preferred_element_type=jnp.float32)
        h = 0.5 * h * (1.0 + lax.erf(h * (2.0 ** -0.5)))
        yb = jnp.dot(h, w2_ref[0], preferred_element_type=jnp.float32)
        ys_ref[...] = yb * sw_ref[0, 0, :][:, None]


def _ffn(be, xs, W1, W2, sw3):
    def real(b, be):
        return b < be[G]

    def expert(b, be):
        return be[jnp.minimum(b, jnp.maximum(be[G] - 1, 0))]

    grid_spec = pltpu.PrefetchScalarGridSpec(
        num_scalar_prefetch=1,
        grid=(G,),
        in_specs=[
            pl.BlockSpec((BLK, D),
                         lambda b, be: (jnp.where(real(b, be), b, 0), 0)),
            pl.BlockSpec((1, D, FF), lambda b, be: (expert(b, be), 0, 0)),
            pl.BlockSpec((1, FF, D), lambda b, be: (expert(b, be), 0, 0)),
            pl.BlockSpec((1, 1, BLK),
                         lambda b, be: (jnp.where(real(b, be), b, 0), 0, 0)),
        ],
        out_specs=pl.BlockSpec(
            (BLK, D), lambda b, be: (jnp.where(real(b, be), b, G), 0)),
    )
    return pl.pallas_call(
        _ffn_body, grid_spec=grid_spec,
        out_shape=jax.ShapeDtypeStruct(((G + 1) * BLK, D), jnp.float32),
    )(be, xs, W1, W2, sw3)


def _combine_body(ys_hbm, d1_hbm, d2_hbm, out_hbm, d1v, d2v, rb1, rb2, sem):
    wid = lax.axis_index("s") * NC + lax.axis_index("c")
    base = wid * TPB
    pltpu.sync_copy(d1_hbm.at[pl.ds(base, TPB)], d1v)
    pltpu.sync_copy(d2_hbm.at[pl.ds(base, TPB)], d2v)
    cp1 = pltpu.async_copy(ys_hbm.at[d1v], rb1, sem)
    cp2 = pltpu.async_copy(ys_hbm.at[d2v], rb2, sem)
    cp1.wait()
    cp2.wait()

    def ab(i, carry):
        r = i // (D // 16)
        c = (i % (D // 16)) * 16
        rb1[r, pl.ds(c, 16)] = rb1[r, pl.ds(c, 16)] + rb2[r, pl.ds(c, 16)]
        return carry

    lax.fori_loop(0, TPB * (D // 16), ab, 0)
    pltpu.sync_copy(rb1, out_hbm.at[pl.ds(base, TPB)])


def _combine(ys, dest1, dest2):
    mesh = plsc.VectorSubcoreMesh(core_axis_name="c", subcore_axis_name="s",
                                  num_cores=NC, num_subcores=NS)
    fn = pl.kernel(
        _combine_body,
        compiler_params=_SC_PARAMS,
        out_type=jax.ShapeDtypeStruct((N, D), jnp.float32),
        mesh=mesh,
        scratch_types=[
            pltpu.VMEM((TPB,), jnp.int32),
            pltpu.VMEM((TPB,), jnp.int32),
            pltpu.VMEM((TPB, D), jnp.float32),
            pltpu.VMEM((TPB, D), jnp.float32),
            pltpu.SemaphoreType.DMA,
        ],
    )
    return fn(ys, dest1, dest2)


def kernel(hidden_states, gate_w, W1, W2):
    x = hidden_states.reshape(N, D)
    u = jax.random.uniform(jax.random.key(42), (N, E), dtype=jnp.float32)
    gum = -jnp.log(-jnp.log(jnp.clip(u, 1e-20, None)) + 1e-20)
    d1, d2, wa1, wa2, be = _routing(x, gate_w.T, gum)
    d1, d2 = d1.reshape(N), d2.reshape(N)
    wa1, wa2, be = wa1.reshape(N), wa2.reshape(N), be.reshape(GP)
    xs, sw = _dispatch(x, d1, d2, wa1, wa2)
    ys = _ffn(be, xs, W1, W2, sw.reshape(G, 1, BLK))
    out = _combine(ys, d1, d2)
    return out.reshape(B, S, D)


# speedup vs baseline: 1.3907x; 1.0010x over previous
import jax
import jax.numpy as jnp
from jax import lax
from jax.experimental import pallas as pl
from jax.experimental.pallas import tpu as pltpu
from jax.experimental.pallas import tpu_sc as plsc

B, S, D, FF, E = 1, 2048, 768, 2048, 64
N = B * S
BLK = 128
G = N * 2 // BLK + E
P = G * BLK
CH = 128
NC, NS = 2, 16
NW = NC * NS
RPT = P // NW
GP = G + 8
GCH = 64
NGC = RPT // GCH
TPB = N // NW


def _routing_body(x_ref, gwt_ref, gum_ref, d1_ref, d2_ref, w1_ref, w2_ref,
                  be_ref):
    x = x_ref[...]
    logits = jnp.dot(x, gwt_ref[...], preferred_element_type=jnp.float32)
    m = jnp.max(logits, axis=-1, keepdims=True)
    eg = jnp.exp(logits - m)
    gates = eg / jnp.sum(eg, axis=-1, keepdims=True)
    iota_e = lax.broadcasted_iota(jnp.int32, (N, E), 1)
    gmax = jnp.max(gates, axis=-1, keepdims=True)
    idx1 = jnp.min(jnp.where(gates == gmax, iota_e, E), axis=-1)
    mask1 = iota_e == idx1[:, None]
    pert = jnp.where(mask1, -jnp.inf, logits + gum_ref[...])
    pmax = jnp.max(pert, axis=-1, keepdims=True)
    idx2 = jnp.min(jnp.where(pert == pmax, iota_e, E), axis=-1)
    mask2 = iota_e == idx2[:, None]
    g1 = jnp.sum(jnp.where(mask1, gates, 0.0), axis=-1)
    g2 = jnp.sum(jnp.where(mask2, gates, 0.0), axis=-1)
    den = jnp.maximum(g1 + g2, jnp.finfo(jnp.float32).eps)
    w1_ref[0, :] = g1 / den
    w2_ref[0, :] = g2 / den

    m1f = mask1.astype(jnp.float32)
    m2f = mask2.astype(jnp.float32)
    tri = (lax.broadcasted_iota(jnp.int32, (CH, CH), 0)
           >= lax.broadcasted_iota(jnp.int32, (CH, CH), 1)).astype(jnp.float32)

    def cumsum_tokens(mf):
        outs = []
        carry = jnp.zeros((1, E), jnp.float32)
        for k in range(N // CH):
            cs = jnp.dot(tri, mf[k * CH:(k + 1) * CH, :],
                         precision=lax.Precision.HIGHEST) + carry
            outs.append(cs)
            carry = cs[CH - 1:CH, :]
        return jnp.concatenate(outs, axis=0)

    cum1 = cumsum_tokens(m1f)
    cum2 = cumsum_tokens(m2f)
    c1 = cum1[N - 1:N, :]
    c2 = cum2[N - 1:N, :]
    r1 = jnp.sum(jnp.where(mask1, cum1, 0.0), axis=-1) - 1.0
    r2 = jnp.sum(jnp.where(mask2, cum2, 0.0), axis=-1) - 1.0
    cnt = c1 + c2
    nb = jnp.ceil(cnt * (1.0 / BLK))
    upper = (lax.broadcasted_iota(jnp.int32, (E, E), 0)
             <= lax.broadcasted_iota(jnp.int32, (E, E), 1)).astype(jnp.float32)
    cumnb = jnp.dot(nb, upper, precision=lax.Precision.HIGHEST)
    poff = (cumnb - nb) * float(BLK)
    dest1 = jnp.sum(jnp.where(mask1, poff, 0.0), axis=-1) + r1
    dest2 = jnp.sum(jnp.where(mask2, poff + c1, 0.0), axis=-1) + r2
    d1_ref[0, :] = jnp.round(dest1).astype(jnp.int32)
    d2_ref[0, :] = jnp.round(dest2).astype(jnp.int32)
    cumnb_i = jnp.round(cumnb).astype(jnp.int32)
    bio = lax.broadcasted_iota(jnp.int32, (GP, E), 0)
    bev = jnp.sum((bio >= cumnb_i).astype(jnp.int32), axis=-1)
    nb_tot = cumnb_i[:, E - 1]
    be_ref[0, :] = jnp.where(bio[:, 0] < G, jnp.minimum(bev, E - 1), nb_tot)


def _routing(x, gwt, gum):
    out_shape = (
        jax.ShapeDtypeStruct((1, N), jnp.int32),
        jax.ShapeDtypeStruct((1, N), jnp.int32),
        jax.ShapeDtypeStruct((1, N), jnp.float32),
        jax.ShapeDtypeStruct((1, N), jnp.float32),
        jax.ShapeDtypeStruct((1, GP), jnp.int32),
    )
    return pl.pallas_call(_routing_body, out_shape=out_shape)(x, gwt, gum)


APT = 2 * N // NS
ZPT = P // NS


def _dispatch_body(x_hbm, d1_hbm, d2_hbm, wa1_hbm, wa2_hbm, xs_hbm, sw_hbm,
                   dvec, tokv, wv, zbi, zbf, gwin, rowbuf,
                   g_sh, sw_sh, gs0, gs1, ws0, ws1):
    cid = lax.axis_index("c")
    sid = lax.axis_index("s")
    wid = cid * NS + sid
    iota16 = lax.broadcasted_iota(jnp.int32, (16,), 0)

    for k in range(ZPT // 16):
        zbi[pl.ds(k * 16, 16)] = (iota16 + (sid * ZPT + k * 16)) & (N - 1)
        zbf[pl.ds(k * 16, 16)] = jnp.zeros((16,), jnp.float32)
    pltpu.sync_copy(zbi, g_sh.at[pl.ds(sid * ZPT, ZPT)])
    pltpu.sync_copy(zbf, sw_sh.at[pl.ds(sid * ZPT, ZPT)])
    plsc.subcore_barrier()

    tbase = (sid % (NS // 2)) * APT

    @pl.when(sid < NS // 2)
    def _():
        for j in range(APT // 128):
            pltpu.sync_copy(d1_hbm.at[pl.ds(tbase + j * 128, 128)],
                            dvec.at[j])
            pltpu.sync_copy(wa1_hbm.at[pl.ds(tbase + j * 128, 128)],
                            wv.at[j])

    @pl.when(sid >= NS // 2)
    def _():
        for j in range(APT // 128):
            pltpu.sync_copy(d2_hbm.at[pl.ds(tbase + j * 128, 128)],
                            dvec.at[j])
            pltpu.sync_copy(wa2_hbm.at[pl.ds(tbase + j * 128, 128)],
                            wv.at[j])

    for j in range(APT // 128):
        for k in range(8):
            tokv[j, pl.ds(k * 16, 16)] = iota16 + (tbase + j * 128 + k * 16)
    for j in range(APT // 128):
        pltpu.sync_copy(tokv.at[j], g_sh.at[dvec.at[j]])
        pltpu.sync_copy(wv.at[j], sw_sh.at[dvec.at[j]])
    plsc.subcore_barrier()

    win = wid * RPT
    for ci in range(NGC):
        pltpu.sync_copy(g_sh.at[pl.ds(win + ci * GCH, GCH)], gwin.at[ci])
    pltpu.sync_copy(sw_sh.at[pl.ds(win, RPT)], sw_hbm.at[pl.ds(win, RPT)])
    gsem = (gs0, gs1)
    wsem = (ws0, ws1)
    gd = [None, None]
    wd = [None, None]
    gd[0] = pltpu.async_copy(x_hbm.at[gwin.at[0]], rowbuf.at[0], gs0)
    for ci in range(NGC):
        b = ci % 2
        gd[b].wait()
        if ci + 1 < NGC:
            if wd[1 - b] is not None:
                wd[1 - b].wait()
            gd[1 - b] = pltpu.async_copy(x_hbm.at[gwin.at[ci + 1]],
                                         rowbuf.at[1 - b], gsem[1 - b])
        wd[b] = pltpu.async_copy(rowbuf.at[b],
                                 xs_hbm.at[pl.ds(win + ci * GCH, GCH)],
                                 wsem[b])
    for b in range(2):
        if wd[b] is not None:
            wd[b].wait()


_SC_PARAMS = pltpu.CompilerParams(needs_layout_passes=False)


def _dispatch(x, dest1, dest2, wa1, wa2):
    mesh = plsc.VectorSubcoreMesh(core_axis_name="c", subcore_axis_name="s",
                                  num_cores=NC, num_subcores=NS)
    fn = pl.kernel(
        _dispatch_body,
        compiler_params=_SC_PARAMS,
        out_type=(jax.ShapeDtypeStruct((P, D), jnp.float32),
                  jax.ShapeDtypeStruct((P,), jnp.float32)),
        mesh=mesh,
        scratch_types=[
            pltpu.VMEM((APT // 128, 128), jnp.int32),
            pltpu.VMEM((APT // 128, 128), jnp.int32),
            pltpu.VMEM((APT // 128, 128), jnp.float32),
            pltpu.VMEM((ZPT,), jnp.int32),
            pltpu.VMEM((ZPT,), jnp.float32),
            pltpu.VMEM((NGC, GCH), jnp.int32),
            pltpu.VMEM((2, GCH, D), jnp.float32),
            pltpu.VMEM_SHARED((P,), jnp.int32),
            pltpu.VMEM_SHARED((P,), jnp.float32),
            pltpu.SemaphoreType.DMA,
            pltpu.SemaphoreType.DMA,
            pltpu.SemaphoreType.DMA,
            pltpu.SemaphoreType.DMA,
        ],
    )
    return fn(x, dest1, dest2, wa1, wa2)


def _ffn_body(be_ref, xs_ref, w1_ref, w2_ref, sw_ref, ys_ref):
    @pl.when(pl.program_id(0) < be_ref[G])
    def _():
        xb = xs_ref[...]
        h = jnp.dot(xb, w1_ref[0], preferred_element_type=jnp.float32)
        h = 0.5 * h * (1.0 + lax.erf(h * (2.0 ** -0.5)))
        yb = jnp.dot(h, w2_ref[0], preferred_element_type=jnp.float32)
        ys_ref[...] = yb * sw_ref[0, 0, :][:, None]


def _ffn(be, xs, W1, W2, sw3):
    def real(b, be):
        return b < be[G]

    def expert(b, be):
        return be[jnp.minimum(b, jnp.maximum(be[G] - 1, 0))]

    grid_spec = pltpu.PrefetchScalarGridSpec(
        num_scalar_prefetch=1,
        grid=(G,),
        in_specs=[
            pl.BlockSpec((BLK, D),
                         lambda b, be: (jnp.where(real(b, be), b, 0), 0)),
            pl.BlockSpec((1, D, FF), lambda b, be: (expert(b, be), 0, 0)),
            pl.BlockSpec((1, FF, D), lambda b, be: (expert(b, be), 0, 0)),
            pl.BlockSpec((1, 1, BLK),
                         lambda b, be: (jnp.where(real(b, be), b, 0), 0, 0)),
        ],
        out_specs=pl.BlockSpec(
            (BLK, D), lambda b, be: (jnp.where(real(b, be), b, G), 0)),
    )
    return pl.pallas_call(
        _ffn_body, grid_spec=grid_spec,
        out_shape=jax.ShapeDtypeStruct(((G + 1) * BLK, D), jnp.float32),
    )(be, xs, W1, W2, sw3)


def _combine_body(ys_hbm, d1_hbm, d2_hbm, out_hbm, d1v, d2v, rb1, rb2, sem):
    wid = lax.axis_index("s") * NC + lax.axis_index("c")
    base = wid * TPB
    pltpu.sync_copy(d1_hbm.at[pl.ds(base, TPB)], d1v)
    pltpu.sync_copy(d2_hbm.at[pl.ds(base, TPB)], d2v)
    cp1 = pltpu.async_copy(ys_hbm.at[d1v], rb1, sem)
    cp2 = pltpu.async_copy(ys_hbm.at[d2v], rb2, sem)
    cp1.wait()
    cp2.wait()

    def ab(i, carry):
        r = i // (D // 16)
        c = (i % (D // 16)) * 16
        rb1[r, pl.ds(c, 16)] = rb1[r, pl.ds(c, 16)] + rb2[r, pl.ds(c, 16)]
        return carry

    lax.fori_loop(0, TPB * (D // 16), ab, 0)
    pltpu.sync_copy(rb1, out_hbm.at[pl.ds(base, TPB)])


def _combine(ys, dest1, dest2):
    mesh = plsc.VectorSubcoreMesh(core_axis_name="c", subcore_axis_name="s",
                                  num_cores=NC, num_subcores=NS)
    fn = pl.kernel(
        _combine_body,
        compiler_params=_SC_PARAMS,
        out_type=jax.ShapeDtypeStruct((N, D), jnp.float32),
        mesh=mesh,
        scratch_types=[
            pltpu.VMEM((TPB,), jnp.int32),
            pltpu.VMEM((TPB,), jnp.int32),
            pltpu.VMEM((TPB, D), jnp.float32),
            pltpu.VMEM((TPB, D), jnp.float32),
            pltpu.SemaphoreType.DMA,
        ],
    )
    return fn(ys, dest1, dest2)


def kernel(hidden_states, gate_w, W1, W2):
    x = hidden_states.reshape(N, D)
    u = jax.random.uniform(jax.random.key(42), (N, E), dtype=jnp.float32)
    gum = -jnp.log(-jnp.log(jnp.clip(u, 1e-20, None)) + 1e-20)
    d1, d2, wa1, wa2, be = _routing(x, gate_w.T, gum)
    d1, d2 = d1.reshape(N), d2.reshape(N)
    wa1, wa2, be = wa1.reshape(N), wa2.reshape(N), be.reshape(GP)
    xs, sw = _dispatch(x, d1, d2, wa1, wa2)
    ys = _ffn(be, xs, W1, W2, sw.reshape(G, 1, BLK))
    out = _combine(ys, d1, d2)
    return out.reshape(B, S, D)
